# baseline (speedup 1.0000x reference)
import jax
import jax.numpy as jnp
from jax.experimental import pallas as pl
from jax.experimental.pallas import tpu as pltpu

F32 = jnp.float32
HH = 128
WW = 128
HWS = HH * WW
TR = 16
TILE = TR * WW
NT = HH // TR
NEG_INF = float("-inf")
POS_INF = float("inf")
BIGI = 1 << 30
BF16 = jnp.bfloat16


def _dot1(w, x):
    return jax.lax.dot_general(w.astype(BF16), x.astype(BF16),
                               (((1,), (0,)), ((), ())),
                               preferred_element_type=F32)


def _shifted_slice(ht, s, width):
    if s < 0:
        return jnp.concatenate(
            [jnp.zeros((64, -s), F32), ht[:, :TILE + s]], axis=1)
    if s + TILE > width:
        ext = s + TILE - width
        return jnp.concatenate(
            [ht[:, s:width], jnp.zeros((64, ext), F32)], axis=1)
    return ht[:, s:s + TILE]


def _conv1_kernel(x_ref, w_ref, h_ref, st_ref):
    w = w_ref[...]
    ssum = jnp.zeros((64, 1), F32)
    ssq = jnp.zeros((64, 1), F32)
    for i in range(NT):
        base = i * TILE
        lo = max(0, base - WW)
        hi = min(HWS, base + TILE + WW)
        width = hi - lo
        h9 = _dot1(w, x_ref[0, :, lo:hi])
        lane = jax.lax.broadcasted_iota(jnp.int32, (64, width), 1) % WW
        mask_d0 = lane != (WW - 1)
        mask_d2 = lane != 0
        out = None
        for dy in range(3):
            for dx in range(3):
                t = dy * 3 + dx
                ht = h9[t * 64:(t + 1) * 64, :]
                if dx == 0:
                    ht = jnp.where(mask_d0, ht, 0.0)
                elif dx == 2:
                    ht = jnp.where(mask_d2, ht, 0.0)
                s = base + (dy - 1) * WW + (dx - 1) - lo
                sl = _shifted_slice(ht, s, width)
                out = sl if out is None else out + sl
        h_ref[0, :, base:base + TILE] = out
        ssum = ssum + jnp.sum(out, axis=1, keepdims=True)
        ssq = ssq + jnp.sum(out * out, axis=1, keepdims=True)
    st_ref[0] = jnp.concatenate([ssum, ssq], axis=1)


def _conv2_kernel(h1_ref, w_ref, sc_ref, h2_ref, st_ref):
    w = w_ref[...]
    scale = sc_ref[:, 0:1]
    shift = sc_ref[:, 1:2]
    z1 = jnp.zeros((64, 1), BF16)
    ssum = jnp.zeros((64, 1), F32)
    ssq = jnp.zeros((64, 1), F32)
    for i in range(NT):
        base = i * TILE
        lo = max(0, base - WW)
        hi = min(HWS, base + TILE + WW)
        width = hi - lo
        lane = jax.lax.broadcasted_iota(jnp.int32, (64, width), 1) % WW
        a1t = jnp.maximum(h1_ref[0, :, lo:hi] * scale + shift, 0.0)
        a1b = a1t.astype(BF16)
        zb = jnp.zeros((64, width), BF16)
        p0 = jnp.where(lane == 0, zb,
                       jnp.concatenate([z1, a1b[:, :width - 1]], axis=1))
        p2 = jnp.where(lane == (WW - 1), zb,
                       jnp.concatenate([a1b[:, 1:], z1], axis=1))
        patches = jnp.concatenate([p0, a1b, p2], axis=0)
        o3 = _dot1(w, patches)
        out = None
        for dy in range(3):
            s = base + (dy - 1) * WW - lo
            sl = _shifted_slice(o3[dy * 64:(dy + 1) * 64, :], s, width)
            out = sl if out is None else out + sl
        h2_ref[0, :, base:base + TILE] = out
        ssum = ssum + jnp.sum(out, axis=1, keepdims=True)
        ssq = ssq + jnp.sum(out * out, axis=1, keepdims=True)
    st_ref[0] = jnp.concatenate([ssum, ssq], axis=1)


def _conv3_kernel(h2_ref, prm_ref, hm_ref, lg_ref, pk_ref):
    prm = prm_ref[...]
    scale = prm[:, 0:1]
    shift = prm[:, 1:2]
    w3 = prm[:, 2:3]
    b3 = prm[0, 3]
    a2 = jnp.maximum(h2_ref[0] * scale + shift, 0.0)
    a2b = a2.astype(BF16).astype(F32)
    w3b = w3.astype(BF16).astype(F32)
    logits = (jnp.sum(a2b * w3b, axis=0) + b3).reshape(HH, WW)
    heat = jax.nn.sigmoid(logits)
    lg_ref[0, 0] = logits
    hm_ref[0, 0] = heat
    rm = heat
    for d in range(1, 5):
        pad = jnp.full((HH, d), NEG_INF)
        left = jnp.concatenate([heat[:, d:], pad], axis=1)
        right = jnp.concatenate([pad, heat[:, :WW - d]], axis=1)
        rm = jnp.maximum(rm, jnp.maximum(left, right))
    cm = rm
    for d in range(1, 5):
        pad = jnp.full((d, WW), NEG_INF)
        up = jnp.concatenate([rm[d:, :], pad], axis=0)
        down = jnp.concatenate([pad, rm[:HH - d, :]], axis=0)
        cm = jnp.maximum(cm, jnp.maximum(up, down))
    is_peak = (heat == cm) & (heat > 0.1)
    n = jnp.sum(is_peak.astype(jnp.int32))
    lin = (jax.lax.broadcasted_iota(jnp.int32, (HH, WW), 0) * WW
           + jax.lax.broadcasted_iota(jnp.int32, (HH, WW), 1))
    v = jnp.where(is_peak, heat, NEG_INF)
    idxs = []
    for _ in range(3):
        m = jnp.max(v)
        idx = jnp.min(jnp.where(v == m, lin, BIGI))
        idxs.append(idx)
        v = jnp.where(lin == idx, NEG_INF, v)
    fallback = jnp.min(jnp.where(heat == jnp.max(heat), lin, BIGI))
    p0 = jnp.where(n == 0, fallback, idxs[0])
    n_pos = jnp.where(n == 0, 1, jnp.minimum(n, 3))
    v2 = jnp.where(lin == p0, POS_INF, heat)
    v2 = jnp.where((lin == idxs[1]) & (n_pos > 1), POS_INF, v2)
    v2 = jnp.where((lin == idxs[2]) & (n_pos > 2), POS_INF, v2)
    negs = []
    for _ in range(2):
        m = jnp.min(v2)
        nidx = jnp.min(jnp.where(v2 == m, lin, BIGI))
        negs.append(nidx)
        v2 = jnp.where(lin == nidx, POS_INF, v2)
    li = jax.lax.broadcasted_iota(jnp.int32, (1, 8), 1)
    row = jnp.where(li == 0, p0,
          jnp.where(li == 1, idxs[1],
          jnp.where(li == 2, idxs[2],
          jnp.where(li == 3, n,
          jnp.where(li == 4, negs[0],
          jnp.where(li == 5, negs[1], 0))))))
    pk_ref[0] = row


def _bn_affine(st, g, b, eps=1e-5):
    n = jnp.float32(4 * HWS)
    ssum = jnp.sum(st[:, :, 0], axis=0)
    ssq = jnp.sum(st[:, :, 1], axis=0)
    mean = ssum / n
    var = ssq / n - mean * mean
    scale = g / jnp.sqrt(var + eps)
    shift = b - mean * scale
    return jnp.stack([scale, shift], axis=1)


def kernel(encoder_features, conv1_w, bn1_g, bn1_b, conv2_w, bn2_g, bn2_b,
           conv3_w, conv3_b):
    B = encoder_features.shape[0]
    x = encoder_features.reshape(B, 256, HWS).astype(BF16)
    w1r = conv1_w.transpose(2, 3, 0, 1).reshape(576, 256).astype(BF16)
    w2r = conv2_w.transpose(2, 0, 3, 1).reshape(192, 192).astype(BF16)

    h1, st1 = pl.pallas_call(
        _conv1_kernel,
        grid=(B,),
        in_specs=[
            pl.BlockSpec((1, 256, HWS), lambda b: (b, 0, 0)),
            pl.BlockSpec((576, 256), lambda b: (0, 0)),
        ],
        out_specs=[
            pl.BlockSpec((1, 64, HWS), lambda b: (b, 0, 0)),
            pl.BlockSpec((1, 64, 2), lambda b: (b, 0, 0)),
        ],
        out_shape=[
            jax.ShapeDtypeStruct((B, 64, HWS), F32),
            jax.ShapeDtypeStruct((B, 64, 2), F32),
        ],
    )(x, w1r)

    sc1 = _bn_affine(st1, bn1_g, bn1_b)

    h2, st2 = pl.pallas_call(
        _conv2_kernel,
        grid=(B,),
        in_specs=[
            pl.BlockSpec((1, 64, HWS), lambda b: (b, 0, 0)),
            pl.BlockSpec((192, 192), lambda b: (0, 0)),
            pl.BlockSpec((64, 2), lambda b: (0, 0)),
        ],
        out_specs=[
            pl.BlockSpec((1, 64, HWS), lambda b: (b, 0, 0)),
            pl.BlockSpec((1, 64, 2), lambda b: (b, 0, 0)),
        ],
        out_shape=[
            jax.ShapeDtypeStruct((B, 64, HWS), F32),
            jax.ShapeDtypeStruct((B, 64, 2), F32),
        ],
    )(h1, w2r, sc1)

    sc2 = _bn_affine(st2, bn2_g, bn2_b)
    prm = jnp.concatenate(
        [sc2, conv3_w.reshape(64, 1),
         jnp.broadcast_to(conv3_b.reshape(1, 1), (64, 1))], axis=1)

    hm, lg, pk = pl.pallas_call(
        _conv3_kernel,
        grid=(B,),
        in_specs=[
            pl.BlockSpec((1, 64, HWS), lambda b: (b, 0, 0)),
            pl.BlockSpec((64, 4), lambda b: (0, 0)),
        ],
        out_specs=[
            pl.BlockSpec((1, 1, HH, WW), lambda b: (b, 0, 0, 0)),
            pl.BlockSpec((1, 1, HH, WW), lambda b: (b, 0, 0, 0)),
            pl.BlockSpec((1, 1, 8), lambda b: (b, 0, 0)),
        ],
        out_shape=[
            jax.ShapeDtypeStruct((B, 1, HH, WW), F32),
            jax.ShapeDtypeStruct((B, 1, HH, WW), F32),
            jax.ShapeDtypeStruct((B, 1, 8), jnp.int32),
        ],
    )(h2, prm)

    p = pk[:, 0, :3]
    n = pk[:, 0, 3]
    ng = pk[:, 0, 4:6]
    n_pos = jnp.where(n == 0, 1, jnp.minimum(n, 3))
    pos = jnp.stack([(p % WW).astype(F32), (p // WW).astype(F32)], axis=-1)
    neg = jnp.stack([(ng % WW).astype(F32), (ng // WW).astype(F32)], axis=-1)
    slot = jnp.arange(5)
    is_pos = slot[None, :] < n_pos[:, None]
    is_neg = (slot[None, :] >= n_pos[:, None]) & (
        slot[None, :] < n_pos[:, None] + 2)
    pos_g = pos[:, jnp.clip(slot, 0, 2)]
    neg_idx = jnp.clip(slot[None, :] - n_pos[:, None], 0, 1)
    neg_g = jnp.take_along_axis(neg, neg_idx[:, :, None], axis=1)
    coords = jnp.where(is_pos[:, :, None], pos_g,
                       jnp.where(is_neg[:, :, None], neg_g, 0.0))
    labels = jnp.where(is_pos, 1, jnp.where(is_neg, 0, -1)).astype(jnp.int32)
    return hm, coords[:, None].astype(F32), labels[:, None], lg


# speedup vs baseline: 1.0817x; 1.0817x over previous
import jax
import jax.numpy as jnp
from jax.experimental import pallas as pl
from jax.experimental.pallas import tpu as pltpu

F32 = jnp.float32
HH = 128
WW = 128
HWS = HH * WW
TR = 16
TILE = TR * WW
NT = HH // TR
NEG_INF = float("-inf")
POS_INF = float("inf")
BIGI = 1 << 30
BF16 = jnp.bfloat16


def _dot1(w, x):
    return jax.lax.dot_general(w.astype(BF16), x.astype(BF16),
                               (((1,), (0,)), ((), ())),
                               preferred_element_type=F32)


def _shifted_slice(ht, s, width):
    if s < 0:
        return jnp.concatenate(
            [jnp.zeros((64, -s), F32), ht[:, :TILE + s]], axis=1)
    if s + TILE > width:
        ext = s + TILE - width
        return jnp.concatenate(
            [ht[:, s:width], jnp.zeros((64, ext), F32)], axis=1)
    return ht[:, s:s + TILE]


def _conv1_kernel(x_ref, w_ref, h_ref, st_ref):
    w = w_ref[...]
    ssum = jnp.zeros((64, 1), F32)
    ssq = jnp.zeros((64, 1), F32)
    for i in range(NT):
        base = i * TILE
        lo = max(0, base - WW)
        hi = min(HWS, base + TILE + WW)
        width = hi - lo
        h9 = _dot1(w, x_ref[0, :, lo:hi])
        lane = jax.lax.broadcasted_iota(jnp.int32, (64, width), 1) % WW
        mask_d0 = lane != (WW - 1)
        mask_d2 = lane != 0
        out = None
        for dy in range(3):
            for dx in range(3):
                t = dy * 3 + dx
                ht = h9[t * 64:(t + 1) * 64, :]
                if dx == 0:
                    ht = jnp.where(mask_d0, ht, 0.0)
                elif dx == 2:
                    ht = jnp.where(mask_d2, ht, 0.0)
                s = base + (dy - 1) * WW + (dx - 1) - lo
                sl = _shifted_slice(ht, s, width)
                out = sl if out is None else out + sl
        h_ref[0, :, base:base + TILE] = out
        ssum = ssum + jnp.sum(out, axis=1, keepdims=True)
        ssq = ssq + jnp.sum(out * out, axis=1, keepdims=True)
    st_ref[0] = jnp.concatenate([ssum, ssq], axis=1)


def _conv2_kernel(h1_ref, w_ref, sc_ref, h2_ref, st_ref):
    w = w_ref[...]
    scale = sc_ref[:, 0:1]
    shift = sc_ref[:, 1:2]
    z1 = jnp.zeros((64, 1), BF16)
    ssum = jnp.zeros((64, 1), F32)
    ssq = jnp.zeros((64, 1), F32)
    for i in range(NT):
        base = i * TILE
        lo = max(0, base - WW)
        hi = min(HWS, base + TILE + WW)
        width = hi - lo
        lane = jax.lax.broadcasted_iota(jnp.int32, (64, width), 1) % WW
        a1t = jnp.maximum(h1_ref[0, :, lo:hi] * scale + shift, 0.0)
        a1b = a1t.astype(BF16)
        zb = jnp.zeros((64, width), BF16)
        p0 = jnp.where(lane == 0, zb,
                       jnp.concatenate([z1, a1b[:, :width - 1]], axis=1))
        p2 = jnp.where(lane == (WW - 1), zb,
                       jnp.concatenate([a1b[:, 1:], z1], axis=1))
        patches = jnp.concatenate([p0, a1b, p2], axis=0)
        o3 = _dot1(w, patches)
        out = None
        for dy in range(3):
            s = base + (dy - 1) * WW - lo
            sl = _shifted_slice(o3[dy * 64:(dy + 1) * 64, :], s, width)
            out = sl if out is None else out + sl
        h2_ref[0, :, base:base + TILE] = out
        ssum = ssum + jnp.sum(out, axis=1, keepdims=True)
        ssq = ssq + jnp.sum(out * out, axis=1, keepdims=True)
    st_ref[0] = jnp.concatenate([ssum, ssq], axis=1)


def _conv3_kernel(h2_ref, prm_ref, hm_ref, lg_ref, pk_ref):
    prm = prm_ref[...]
    scale = prm[:, 0:1]
    shift = prm[:, 1:2]
    w3 = prm[:, 2:3]
    b3 = prm[0, 3]
    a2 = jnp.maximum(h2_ref[0] * scale + shift, 0.0)
    a2b = a2.astype(BF16).astype(F32)
    w3b = w3.astype(BF16).astype(F32)
    logits = (jnp.sum(a2b * w3b, axis=0) + b3).reshape(HH, WW)
    heat = jax.nn.sigmoid(logits)
    lg_ref[0, 0] = logits
    hm_ref[0, 0] = heat
    rm = heat
    for d in range(1, 5):
        pad = jnp.full((HH, d), NEG_INF)
        left = jnp.concatenate([heat[:, d:], pad], axis=1)
        right = jnp.concatenate([pad, heat[:, :WW - d]], axis=1)
        rm = jnp.maximum(rm, jnp.maximum(left, right))
    cm = rm
    for d in range(1, 5):
        pad = jnp.full((d, WW), NEG_INF)
        up = jnp.concatenate([rm[d:, :], pad], axis=0)
        down = jnp.concatenate([pad, rm[:HH - d, :]], axis=0)
        cm = jnp.maximum(cm, jnp.maximum(up, down))
    is_peak = (heat == cm) & (heat > 0.1)
    n = jnp.sum(is_peak.astype(jnp.int32))
    lin = (jax.lax.broadcasted_iota(jnp.int32, (HH, WW), 0) * WW
           + jax.lax.broadcasted_iota(jnp.int32, (HH, WW), 1))
    v = jnp.where(is_peak, heat, NEG_INF)
    idxs = []
    for _ in range(3):
        m = jnp.max(v)
        idx = jnp.min(jnp.where(v == m, lin, BIGI))
        idxs.append(idx)
        v = jnp.where(lin == idx, NEG_INF, v)
    fallback = jnp.min(jnp.where(heat == jnp.max(heat), lin, BIGI))
    p0 = jnp.where(n == 0, fallback, idxs[0])
    n_pos = jnp.where(n == 0, 1, jnp.minimum(n, 3))
    v2 = jnp.where(lin == p0, POS_INF, heat)
    v2 = jnp.where((lin == idxs[1]) & (n_pos > 1), POS_INF, v2)
    v2 = jnp.where((lin == idxs[2]) & (n_pos > 2), POS_INF, v2)
    negs = []
    for _ in range(2):
        m = jnp.min(v2)
        nidx = jnp.min(jnp.where(v2 == m, lin, BIGI))
        negs.append(nidx)
        v2 = jnp.where(lin == nidx, POS_INF, v2)
    li = jax.lax.broadcasted_iota(jnp.int32, (1, 8), 1)
    row = jnp.where(li == 0, p0,
          jnp.where(li == 1, idxs[1],
          jnp.where(li == 2, idxs[2],
          jnp.where(li == 3, n,
          jnp.where(li == 4, negs[0],
          jnp.where(li == 5, negs[1], 0))))))
    pk_ref[0] = row


def _bn_affine(st, g, b, eps=1e-5):
    n = jnp.float32(4 * HWS)
    ssum = jnp.sum(st[:, :, 0], axis=0)
    ssq = jnp.sum(st[:, :, 1], axis=0)
    mean = ssum / n
    var = ssq / n - mean * mean
    scale = g / jnp.sqrt(var + eps)
    shift = b - mean * scale
    return jnp.stack([scale, shift], axis=1)


def kernel(encoder_features, conv1_w, bn1_g, bn1_b, conv2_w, bn2_g, bn2_b,
           conv3_w, conv3_b):
    B = encoder_features.shape[0]
    x = encoder_features.reshape(B, 256, HWS)
    w1r = conv1_w.transpose(2, 3, 0, 1).reshape(576, 256).astype(BF16)
    w2r = conv2_w.transpose(2, 0, 3, 1).reshape(192, 192).astype(BF16)

    h1, st1 = pl.pallas_call(
        _conv1_kernel,
        grid=(B,),
        in_specs=[
            pl.BlockSpec((1, 256, HWS), lambda b: (b, 0, 0)),
            pl.BlockSpec((576, 256), lambda b: (0, 0)),
        ],
        out_specs=[
            pl.BlockSpec((1, 64, HWS), lambda b: (b, 0, 0)),
            pl.BlockSpec((1, 64, 2), lambda b: (b, 0, 0)),
        ],
        out_shape=[
            jax.ShapeDtypeStruct((B, 64, HWS), F32),
            jax.ShapeDtypeStruct((B, 64, 2), F32),
        ],
    )(x, w1r)

    sc1 = _bn_affine(st1, bn1_g, bn1_b)

    h2, st2 = pl.pallas_call(
        _conv2_kernel,
        grid=(B,),
        in_specs=[
            pl.BlockSpec((1, 64, HWS), lambda b: (b, 0, 0)),
            pl.BlockSpec((192, 192), lambda b: (0, 0)),
            pl.BlockSpec((64, 2), lambda b: (0, 0)),
        ],
        out_specs=[
            pl.BlockSpec((1, 64, HWS), lambda b: (b, 0, 0)),
            pl.BlockSpec((1, 64, 2), lambda b: (b, 0, 0)),
        ],
        out_shape=[
            jax.ShapeDtypeStruct((B, 64, HWS), F32),
            jax.ShapeDtypeStruct((B, 64, 2), F32),
        ],
    )(h1, w2r, sc1)

    sc2 = _bn_affine(st2, bn2_g, bn2_b)
    prm = jnp.concatenate(
        [sc2, conv3_w.reshape(64, 1),
         jnp.broadcast_to(conv3_b.reshape(1, 1), (64, 1))], axis=1)

    hm, lg, pk = pl.pallas_call(
        _conv3_kernel,
        grid=(B,),
        in_specs=[
            pl.BlockSpec((1, 64, HWS), lambda b: (b, 0, 0)),
            pl.BlockSpec((64, 4), lambda b: (0, 0)),
        ],
        out_specs=[
            pl.BlockSpec((1, 1, HH, WW), lambda b: (b, 0, 0, 0)),
            pl.BlockSpec((1, 1, HH, WW), lambda b: (b, 0, 0, 0)),
            pl.BlockSpec((1, 1, 8), lambda b: (b, 0, 0)),
        ],
        out_shape=[
            jax.ShapeDtypeStruct((B, 1, HH, WW), F32),
            jax.ShapeDtypeStruct((B, 1, HH, WW), F32),
            jax.ShapeDtypeStruct((B, 1, 8), jnp.int32),
        ],
    )(h2, prm)

    p = pk[:, 0, :3]
    n = pk[:, 0, 3]
    ng = pk[:, 0, 4:6]
    n_pos = jnp.where(n == 0, 1, jnp.minimum(n, 3))
    pos = jnp.stack([(p % WW).astype(F32), (p // WW).astype(F32)], axis=-1)
    neg = jnp.stack([(ng % WW).astype(F32), (ng // WW).astype(F32)], axis=-1)
    slot = jnp.arange(5)
    is_pos = slot[None, :] < n_pos[:, None]
    is_neg = (slot[None, :] >= n_pos[:, None]) & (
        slot[None, :] < n_pos[:, None] + 2)
    pos_g = pos[:, jnp.clip(slot, 0, 2)]
    neg_idx = jnp.clip(slot[None, :] - n_pos[:, None], 0, 1)
    neg_g = jnp.take_along_axis(neg, neg_idx[:, :, None], axis=1)
    coords = jnp.where(is_pos[:, :, None], pos_g,
                       jnp.where(is_neg[:, :, None], neg_g, 0.0))
    labels = jnp.where(is_pos, 1, jnp.where(is_neg, 0, -1)).astype(jnp.int32)
    return hm, coords[:, None].astype(F32), labels[:, None], lg


# speedup vs baseline: 1.0835x; 1.0016x over previous
import jax
import jax.numpy as jnp
from jax.experimental import pallas as pl

F32 = jnp.float32
HH = 128
WW = 128
HWS = HH * WW
TR = 16
TILE = TR * WW
NT = HH // TR
NEG_INF = float("-inf")
POS_INF = float("inf")
BIGI = 1 << 30
BF16 = jnp.bfloat16


def _dot1(w, x):
    return jax.lax.dot_general(w.astype(BF16), x.astype(BF16),
                               (((1,), (0,)), ((), ())),
                               preferred_element_type=F32)


def _shifted_slice(ht, s, width):
    if s < 0:
        return jnp.concatenate(
            [jnp.zeros((64, -s), F32), ht[:, :TILE + s]], axis=1)
    if s + TILE > width:
        ext = s + TILE - width
        return jnp.concatenate(
            [ht[:, s:width], jnp.zeros((64, ext), F32)], axis=1)
    return ht[:, s:s + TILE]


def _conv1_kernel(x_ref, w_ref, h_ref, st_ref):
    w = w_ref[...]
    ssum = jnp.zeros((64, 1), F32)
    ssq = jnp.zeros((64, 1), F32)
    for i in range(NT):
        base = i * TILE
        lo = max(0, base - WW)
        hi = min(HWS, base + TILE + WW)
        width = hi - lo
        h9 = _dot1(w, x_ref[0, :, lo:hi])
        lane = jax.lax.broadcasted_iota(jnp.int32, (64, width), 1) % WW
        mask_d0 = lane != (WW - 1)
        mask_d2 = lane != 0
        out = None
        for dy in range(3):
            for dx in range(3):
                t = dy * 3 + dx
                ht = h9[t * 64:(t + 1) * 64, :]
                if dx == 0:
                    ht = jnp.where(mask_d0, ht, 0.0)
                elif dx == 2:
                    ht = jnp.where(mask_d2, ht, 0.0)
                s = base + (dy - 1) * WW + (dx - 1) - lo
                sl = _shifted_slice(ht, s, width)
                out = sl if out is None else out + sl
        h_ref[0, :, base:base + TILE] = out
        ssum = ssum + jnp.sum(out, axis=1, keepdims=True)
        ssq = ssq + jnp.sum(out * out, axis=1, keepdims=True)
    st_ref[0] = jnp.concatenate([ssum, ssq], axis=1)


def _conv2_kernel(h1_ref, w_ref, sc_ref, h2_ref, st_ref):
    w = w_ref[...]
    scale = sc_ref[:, 0:1]
    shift = sc_ref[:, 1:2]
    z1 = jnp.zeros((64, 1), BF16)
    ssum = jnp.zeros((64, 1), F32)
    ssq = jnp.zeros((64, 1), F32)
    for i in range(NT):
        base = i * TILE
        lo = max(0, base - WW)
        hi = min(HWS, base + TILE + WW)
        width = hi - lo
        lane = jax.lax.broadcasted_iota(jnp.int32, (64, width), 1) % WW
        a1t = jnp.maximum(h1_ref[0, :, lo:hi] * scale + shift, 0.0)
        a1b = a1t.astype(BF16)
        zb = jnp.zeros((64, width), BF16)
        p0 = jnp.where(lane == 0, zb,
                       jnp.concatenate([z1, a1b[:, :width - 1]], axis=1))
        p2 = jnp.where(lane == (WW - 1), zb,
                       jnp.concatenate([a1b[:, 1:], z1], axis=1))
        patches = jnp.concatenate([p0, a1b, p2], axis=0)
        o3 = _dot1(w, patches)
        out = None
        for dy in range(3):
            s = base + (dy - 1) * WW - lo
            sl = _shifted_slice(o3[dy * 64:(dy + 1) * 64, :], s, width)
            out = sl if out is None else out + sl
        h2_ref[0, :, base:base + TILE] = out
        ssum = ssum + jnp.sum(out, axis=1, keepdims=True)
        ssq = ssq + jnp.sum(out * out, axis=1, keepdims=True)
    st_ref[0] = jnp.concatenate([ssum, ssq], axis=1)


def _conv3_kernel(h2_ref, prm_ref, hm_ref, lg_ref, pk_ref):
    prm = prm_ref[...]
    scale = prm[:, 0:1]
    shift = prm[:, 1:2]
    w3 = prm[:, 2:3]
    b3 = prm[0, 3]
    a2 = jnp.maximum(h2_ref[0] * scale + shift, 0.0)
    a2b = a2.astype(BF16).astype(F32)
    w3b = w3.astype(BF16).astype(F32)
    logits = (jnp.sum(a2b * w3b, axis=0) + b3).reshape(HH, WW)
    heat = jax.nn.sigmoid(logits)
    lg_ref[0, 0] = logits
    hm_ref[0, 0] = heat
    rm = heat
    for d in range(1, 5):
        pad = jnp.full((HH, d), NEG_INF)
        left = jnp.concatenate([heat[:, d:], pad], axis=1)
        right = jnp.concatenate([pad, heat[:, :WW - d]], axis=1)
        rm = jnp.maximum(rm, jnp.maximum(left, right))
    cm = rm
    for d in range(1, 5):
        pad = jnp.full((d, WW), NEG_INF)
        up = jnp.concatenate([rm[d:, :], pad], axis=0)
        down = jnp.concatenate([pad, rm[:HH - d, :]], axis=0)
        cm = jnp.maximum(cm, jnp.maximum(up, down))
    is_peak = (heat == cm) & (heat > 0.1)
    n = jnp.sum(is_peak.astype(jnp.int32))
    lin = (jax.lax.broadcasted_iota(jnp.int32, (HH, WW), 0) * WW
           + jax.lax.broadcasted_iota(jnp.int32, (HH, WW), 1))
    v = jnp.where(is_peak, heat, NEG_INF)
    idxs = []
    for _ in range(3):
        m = jnp.max(v)
        idx = jnp.min(jnp.where(v == m, lin, BIGI))
        idxs.append(idx)
        v = jnp.where(lin == idx, NEG_INF, v)
    fallback = jnp.min(jnp.where(heat == jnp.max(heat), lin, BIGI))
    p0 = jnp.where(n == 0, fallback, idxs[0])
    n_pos = jnp.where(n == 0, 1, jnp.minimum(n, 3))
    v2 = jnp.where(lin == p0, POS_INF, heat)
    v2 = jnp.where((lin == idxs[1]) & (n_pos > 1), POS_INF, v2)
    v2 = jnp.where((lin == idxs[2]) & (n_pos > 2), POS_INF, v2)
    negs = []
    for _ in range(2):
        m = jnp.min(v2)
        nidx = jnp.min(jnp.where(v2 == m, lin, BIGI))
        negs.append(nidx)
        v2 = jnp.where(lin == nidx, POS_INF, v2)
    li = jax.lax.broadcasted_iota(jnp.int32, (1, 8), 1)
    row = jnp.where(li == 0, p0,
          jnp.where(li == 1, idxs[1],
          jnp.where(li == 2, idxs[2],
          jnp.where(li == 3, n,
          jnp.where(li == 4, negs[0],
          jnp.where(li == 5, negs[1], 0))))))
    pk_ref[0] = row


def _bn_affine(st, g, b, eps=1e-5):
    n = jnp.float32(4 * HWS)
    ssum = jnp.sum(st[:, :, 0], axis=0)
    ssq = jnp.sum(st[:, :, 1], axis=0)
    mean = ssum / n
    var = ssq / n - mean * mean
    scale = g / jnp.sqrt(var + eps)
    shift = b - mean * scale
    return jnp.stack([scale, shift], axis=1)


def kernel(encoder_features, conv1_w, bn1_g, bn1_b, conv2_w, bn2_g, bn2_b,
           conv3_w, conv3_b):
    B = encoder_features.shape[0]
    x = encoder_features.reshape(B, 256, HWS)
    w1r = conv1_w.transpose(2, 3, 0, 1).reshape(576, 256).astype(BF16)
    w2r = conv2_w.transpose(2, 0, 3, 1).reshape(192, 192).astype(BF16)

    h1, st1 = pl.pallas_call(
        _conv1_kernel,
        grid=(B,),
        in_specs=[
            pl.BlockSpec((1, 256, HWS), lambda b: (b, 0, 0)),
            pl.BlockSpec((576, 256), lambda b: (0, 0)),
        ],
        out_specs=[
            pl.BlockSpec((1, 64, HWS), lambda b: (b, 0, 0)),
            pl.BlockSpec((1, 64, 2), lambda b: (b, 0, 0)),
        ],
        out_shape=[
            jax.ShapeDtypeStruct((B, 64, HWS), F32),
            jax.ShapeDtypeStruct((B, 64, 2), F32),
        ],
    )(x, w1r)

    sc1 = _bn_affine(st1, bn1_g, bn1_b)

    h2, st2 = pl.pallas_call(
        _conv2_kernel,
        grid=(B,),
        in_specs=[
            pl.BlockSpec((1, 64, HWS), lambda b: (b, 0, 0)),
            pl.BlockSpec((192, 192), lambda b: (0, 0)),
            pl.BlockSpec((64, 2), lambda b: (0, 0)),
        ],
        out_specs=[
            pl.BlockSpec((1, 64, HWS), lambda b: (b, 0, 0)),
            pl.BlockSpec((1, 64, 2), lambda b: (b, 0, 0)),
        ],
        out_shape=[
            jax.ShapeDtypeStruct((B, 64, HWS), F32),
            jax.ShapeDtypeStruct((B, 64, 2), F32),
        ],
    )(h1, w2r, sc1)

    sc2 = _bn_affine(st2, bn2_g, bn2_b)
    prm = jnp.concatenate(
        [sc2, conv3_w.reshape(64, 1),
         jnp.broadcast_to(conv3_b.reshape(1, 1), (64, 1))], axis=1)

    hm, lg, pk = pl.pallas_call(
        _conv3_kernel,
        grid=(B,),
        in_specs=[
            pl.BlockSpec((1, 64, HWS), lambda b: (b, 0, 0)),
            pl.BlockSpec((64, 4), lambda b: (0, 0)),
        ],
        out_specs=[
            pl.BlockSpec((1, 1, HH, WW), lambda b: (b, 0, 0, 0)),
            pl.BlockSpec((1, 1, HH, WW), lambda b: (b, 0, 0, 0)),
            pl.BlockSpec((1, 1, 8), lambda b: (b, 0, 0)),
        ],
        out_shape=[
            jax.ShapeDtypeStruct((B, 1, HH, WW), F32),
            jax.ShapeDtypeStruct((B, 1, HH, WW), F32),
            jax.ShapeDtypeStruct((B, 1, 8), jnp.int32),
        ],
    )(h2, prm)

    p = pk[:, 0, :3]
    n = pk[:, 0, 3]
    ng = pk[:, 0, 4:6]
    n_pos = jnp.where(n == 0, 1, jnp.minimum(n, 3))
    pos = jnp.stack([(p % WW).astype(F32), (p // WW).astype(F32)], axis=-1)
    neg = jnp.stack([(ng % WW).astype(F32), (ng // WW).astype(F32)], axis=-1)
    slot = jnp.arange(5)
    is_pos = slot[None, :] < n_pos[:, None]
    is_neg = (slot[None, :] >= n_pos[:, None]) & (
        slot[None, :] < n_pos[:, None] + 2)
    pos_g = pos[:, jnp.clip(slot, 0, 2)]
    neg_idx = jnp.clip(slot[None, :] - n_pos[:, None], 0, 1)
    neg_g = jnp.take_along_axis(neg, neg_idx[:, :, None], axis=1)
    coords = jnp.where(is_pos[:, :, None], pos_g,
                       jnp.where(is_neg[:, :, None], neg_g, 0.0))
    labels = jnp.where(is_pos, 1, jnp.where(is_neg, 0, -1)).astype(jnp.int32)
    return hm, coords[:, None].astype(F32), labels[:, None], lg
